# baseline (speedup 1.0000x reference)
import functools

import jax
import jax.numpy as jnp
from jax import lax
from jax.experimental import pallas as pl
from jax.experimental.pallas import tpu as pltpu
from jax.experimental.pallas import tpu_sc as plsc

_N = 10000
_E = 320000
_D = 128
_ALPHA = 0.5
_EPS = 1e-5

_NC = 2
_NS = 16
_NW = _NC * _NS
_CH = 128
_CPW0 = 128
_CPW1 = 32
_NCH = _NS * (_CPW0 + _CPW1)
_NCHP = _NCH + 16
_NB = 2
_PB = 16
_EPAD = _NCHP * _CH
_CPD = _NCH // _NW
_ROWS = 10240
_RPT = _ROWS // _NS
_DUMMY = _N
_DEGW = 16
_ZR = 16

_mesh = plsc.VectorSubcoreMesh(core_axis_name="c", subcore_axis_name="s",
                               num_cores=_NC)


def _deg_body(rowp, out, rowv, hist):
    cid = lax.axis_index("c")
    sid = lax.axis_index("s")
    wid = sid * _NC + cid

    def zstep(i, carry):
        hist[pl.ds(i * 16, 16)] = jnp.zeros((16,), jnp.float32)
        return carry

    lax.fori_loop(0, _ROWS // 16, zstep, 0)
    pltpu.sync_copy(rowp.at[pl.ds(wid * _CPD, _CPD)], rowv)
    ones = jnp.ones((16,), jnp.float32)

    def estep(c, carry):
        for k in range(_CH // 16):
            idx = rowv[c, pl.ds(k * 16, 16)]
            plsc.addupdate_scatter(hist, [idx], ones)
        return carry

    lax.fori_loop(0, _CPD, estep, 0)
    pltpu.sync_copy(hist, out.at[wid])


_deg_call = pl.kernel(
    _deg_body,
    out_type=jax.ShapeDtypeStruct((_NW, _ROWS), jnp.float32),
    mesh=_mesh,
    scratch_types=[
        pltpu.VMEM((_CPD, _CH), jnp.int32),
        pltpu.VMEM((_ROWS,), jnp.float32),
    ],
    compiler_params=pltpu.CompilerParams(needs_layout_passes=False),
)


def _agg_body(g, packed, out, pblk, colcur, rowcur, gbuf, zbuf, acc_sh,
              gsem, psem):
    cid = lax.axis_index("c")
    sid = lax.axis_index("s")
    for r in range(_ZR):
        for k in range(_D // 16):
            zbuf[r, pl.ds(k * 16, 16)] = jnp.zeros((16,), jnp.float32)

    def zstep(i, carry):
        pltpu.sync_copy(zbuf, acc_sh.at[pl.ds(sid * _RPT + i * _ZR, _ZR)])
        return carry

    lax.fori_loop(0, _RPT // _ZR, zstep, 0)
    plsc.subcore_barrier()

    def unpack(slot, wi, tgt):
        for k in range(_CH // 16):
            pv = pblk[slot, wi, pl.ds(k * 16, 16)]
            colcur[tgt, pl.ds(k * 16, 16)] = jnp.bitwise_and(pv, 16383)
            rowcur[tgt, pl.ds(k * 16, 16)] = jnp.right_shift(pv, 14)

    def pipeline(cpw, base):
        nblk = (cpw + _PB - 1) // _PB
        pltpu.async_copy(packed.at[pl.ds(base, _PB)], pblk.at[0], psem.at[0])
        if nblk > 1:
            pltpu.async_copy(packed.at[pl.ds(base + _PB, _PB)], pblk.at[1],
                             psem.at[1])
        pltpu.make_async_copy(packed.at[pl.ds(base, _PB)], pblk.at[0],
                              psem.at[0]).wait()
        for c in range(_NB):
            unpack(0, c, c)
            pltpu.async_copy(g.at[colcur.at[c]], gbuf.at[c], gsem.at[c])

        def estep(c, carry):
            bg = lax.rem(c, _NB)
            pltpu.make_async_copy(g.at[colcur.at[bg]], gbuf.at[bg],
                                  gsem.at[bg]).wait()
            pltpu.sync_copy(gbuf.at[bg], acc_sh.at[rowcur.at[bg]], add=True)
            cn = c + _NB
            blk = lax.div(cn, _PB)
            wi = lax.rem(cn, _PB)
            slot = lax.rem(blk, 2)

            @pl.when(wi == 0)
            def _():
                pltpu.make_async_copy(packed.at[pl.ds(base, _PB)],
                                      pblk.at[slot], psem.at[slot]).wait()

                @pl.when((blk + 1) * _PB < cpw)
                def _():
                    pltpu.async_copy(
                        packed.at[pl.ds(base + (blk + 1) * _PB, _PB)],
                        pblk.at[1 - slot], psem.at[1 - slot])

            unpack(slot, wi, bg)
            pltpu.async_copy(g.at[colcur.at[bg]], gbuf.at[bg], gsem.at[bg])
            return carry

        lax.fori_loop(0, cpw - _NB, estep, 0)
        for i in range(_NB):
            c = cpw - _NB + i
            bg = c % _NB
            pltpu.make_async_copy(g.at[colcur.at[bg]], gbuf.at[bg],
                                  gsem.at[bg]).wait()
            pltpu.sync_copy(gbuf.at[bg], acc_sh.at[rowcur.at[bg]], add=True)

    @pl.when(cid == 0)
    def _():
        pipeline(_CPW0, sid * _CPW0)

    @pl.when(cid == 1)
    def _():
        pipeline(_CPW1, _NS * _CPW0 + sid * _CPW1)

    plsc.subcore_barrier()
    pltpu.sync_copy(acc_sh.at[pl.ds(sid * _RPT, _RPT)],
                    out.at[cid, pl.ds(sid * _RPT, _RPT)])


_agg_call = pl.kernel(
    _agg_body,
    out_type=jax.ShapeDtypeStruct((_NC, _ROWS, _D), jnp.float32),
    mesh=_mesh,
    scratch_types=[
        pltpu.VMEM((2, _PB, _CH), jnp.int32),
        pltpu.VMEM((2, _CH), jnp.int32),
        pltpu.VMEM((2, _CH), jnp.int32),
        pltpu.VMEM((_NB, _CH, _D), jnp.float32),
        pltpu.VMEM((_ZR, _D), jnp.float32),
        pltpu.VMEM_SHARED((_ROWS, _D), jnp.float32),
        pltpu.SemaphoreType.DMA((_NB,)),
        pltpu.SemaphoreType.DMA((2,)),
    ],
    compiler_params=pltpu.CompilerParams(needs_layout_passes=False),
)


def _prep_body(degp, x, dinv_ref, g_ref):
    deg = jnp.sum(degp[:, : _N], axis=0).reshape(_N, 1)
    dinv = jnp.where(deg > 0.0, lax.rsqrt(deg), 0.0)
    dinv_ref[...] = dinv
    g_ref[...] = x[...] * dinv


_prep_call = pl.pallas_call(
    _prep_body,
    out_shape=(
        jax.ShapeDtypeStruct((_N, 1), jnp.float32),
        jax.ShapeDtypeStruct((_N, _D), jnp.float32),
    ),
)


def _dense_body(h, aggp, dinv, W, b, gam, bet, hout, gout):
    dv = dinv[...]
    agg = jnp.sum(aggp[:, : _N, :], axis=0)
    t = h[...] - _ALPHA * dv * agg
    z = jnp.dot(t, W[...], preferred_element_type=jnp.float32) + b[...]
    mu = jnp.mean(z, axis=0, keepdims=True)
    zc = z - mu
    var = jnp.mean(zc * zc, axis=0, keepdims=True)
    hn = jnp.maximum(zc * lax.rsqrt(var + _EPS) * gam[...] + bet[...], 0.0)
    hout[...] = hn
    gout[...] = hn * dv


_dense_call = pl.pallas_call(
    _dense_body,
    out_shape=(
        jax.ShapeDtypeStruct((_N, _D), jnp.float32),
        jax.ShapeDtypeStruct((_N, _D), jnp.float32),
    ),
)


def _final_body(h, aggp, dinv, W, b, out):
    agg = jnp.sum(aggp[:, : _N, :], axis=0)
    t = h[...] - _ALPHA * dinv[...] * agg
    out[...] = jnp.dot(t, W[...], preferred_element_type=jnp.float32) + b[...]


_final_call = pl.pallas_call(
    _final_body,
    out_shape=jax.ShapeDtypeStruct((_N, _D), jnp.float32),
)


def kernel(x, edge_index, W1, b1, W2, b2, W3, b3, g1, be1, g2, be2):
    row = edge_index[0]
    col = edge_index[1]
    pad = _EPAD - _E
    rowp = jnp.concatenate(
        [row, jnp.full((pad,), _DUMMY, jnp.int32)]).reshape(_NCHP, _CH)
    colp = jnp.concatenate(
        [col, jnp.zeros((pad,), jnp.int32)]).reshape(_NCHP, _CH)
    packed = jnp.left_shift(rowp, 14) | colp

    degp = _deg_call(rowp)
    dinv, g = _prep_call(degp, x)

    aggp = _agg_call(g, packed)
    h, g = _dense_call(x, aggp, dinv, W1, b1.reshape(1, _D),
                       g1.reshape(1, _D), be1.reshape(1, _D))
    aggp = _agg_call(g, packed)
    h, g = _dense_call(h, aggp, dinv, W2, b2.reshape(1, _D),
                       g2.reshape(1, _D), be2.reshape(1, _D))
    aggp = _agg_call(g, packed)
    return _final_call(h, aggp, dinv, W3, b3.reshape(1, _D))


# speedup vs baseline: 1.3875x; 1.3875x over previous
import functools

import jax
import jax.numpy as jnp
from jax import lax
from jax.experimental import pallas as pl
from jax.experimental.pallas import tpu as pltpu
from jax.experimental.pallas import tpu_sc as plsc

_N = 10000
_E = 320000
_D = 128
_ALPHA = 0.5
_EPS = 1e-5

_NC = 2
_NS = 16
_NW = _NC * _NS
_CH = 128
_CPW = 79
_EPAD = _NW * _CPW * _CH
_ROWS = 10240
_RPT = _ROWS // _NS
_DUMMY = _N
_ZR = 16

_mesh = plsc.VectorSubcoreMesh(core_axis_name="c", subcore_axis_name="s")


def _deg_body(rowp, out, rowv, hist):
    cid = lax.axis_index("c")
    sid = lax.axis_index("s")
    wid = sid * _NC + cid

    def zstep(i, carry):
        hist[pl.ds(i * 16, 16)] = jnp.zeros((16,), jnp.float32)
        return carry

    lax.fori_loop(0, _ROWS // 16, zstep, 0)
    pltpu.sync_copy(rowp.at[wid], rowv)
    ones = jnp.ones((16,), jnp.float32)

    def estep(c, carry):
        for k in range(_CH // 16):
            idx = rowv[c, pl.ds(k * 16, 16)]
            plsc.addupdate_scatter(hist, [idx], ones)
        return carry

    lax.fori_loop(0, _CPW, estep, 0)
    pltpu.sync_copy(hist, out.at[wid])


_deg_call = pl.kernel(
    _deg_body,
    out_type=jax.ShapeDtypeStruct((_NW, _ROWS), jnp.float32),
    mesh=_mesh,
    scratch_types=[
        pltpu.VMEM((_CPW, _CH), jnp.int32),
        pltpu.VMEM((_ROWS,), jnp.float32),
    ],
    compiler_params=pltpu.CompilerParams(needs_layout_passes=False),
)


def _agg_body(g, colp, rowp, out, colv, rowv, gbuf, zbuf, acc_sh, sem):
    cid = lax.axis_index("c")
    sid = lax.axis_index("s")
    wid = sid * _NC + cid
    for r in range(_ZR):
        for k in range(_D // 16):
            zbuf[r, pl.ds(k * 16, 16)] = jnp.zeros((16,), jnp.float32)

    def zstep(i, carry):
        pltpu.sync_copy(zbuf, acc_sh.at[pl.ds(sid * _RPT + i * _ZR, _ZR)])
        return carry

    lax.fori_loop(0, _RPT // _ZR, zstep, 0)
    pltpu.sync_copy(colp.at[wid], colv)
    pltpu.sync_copy(rowp.at[wid], rowv)
    plsc.subcore_barrier()

    def estep(c, carry):
        pltpu.async_copy(g.at[colv.at[c]], gbuf, sem).wait()
        pltpu.sync_copy(gbuf, acc_sh.at[rowv.at[c]], add=True)
        return carry

    lax.fori_loop(0, _CPW, estep, 0)
    plsc.subcore_barrier()
    pltpu.sync_copy(acc_sh.at[pl.ds(sid * _RPT, _RPT)],
                    out.at[cid, pl.ds(sid * _RPT, _RPT)])


_agg_call = pl.kernel(
    _agg_body,
    out_type=jax.ShapeDtypeStruct((_NC, _ROWS, _D), jnp.float32),
    mesh=_mesh,
    scratch_types=[
        pltpu.VMEM((_CPW, _CH), jnp.int32),
        pltpu.VMEM((_CPW, _CH), jnp.int32),
        pltpu.VMEM((_CH, _D), jnp.float32),
        pltpu.VMEM((_ZR, _D), jnp.float32),
        pltpu.VMEM_SHARED((_ROWS, _D), jnp.float32),
        pltpu.SemaphoreType.DMA,
    ],
)


def _prep_body(degp, x, dinv_ref, g_ref):
    deg = jnp.sum(degp[:, : _N], axis=0).reshape(_N, 1)
    dinv = jnp.where(deg > 0.0, lax.rsqrt(deg), 0.0)
    dinv_ref[...] = dinv
    g_ref[...] = x[...] * dinv


_prep_call = pl.pallas_call(
    _prep_body,
    out_shape=(
        jax.ShapeDtypeStruct((_N, 1), jnp.float32),
        jax.ShapeDtypeStruct((_N, _D), jnp.float32),
    ),
)


def _dense_body(h, aggp, dinv, W, b, gam, bet, hout, gout):
    dv = dinv[...]
    agg = jnp.sum(aggp[:, : _N, :], axis=0)
    t = h[...] - _ALPHA * dv * agg
    z = jnp.dot(t, W[...], preferred_element_type=jnp.float32) + b[...]
    mu = jnp.mean(z, axis=0, keepdims=True)
    zc = z - mu
    var = jnp.mean(zc * zc, axis=0, keepdims=True)
    hn = jnp.maximum(zc * lax.rsqrt(var + _EPS) * gam[...] + bet[...], 0.0)
    hout[...] = hn
    gout[...] = hn * dv


_dense_call = pl.pallas_call(
    _dense_body,
    out_shape=(
        jax.ShapeDtypeStruct((_N, _D), jnp.float32),
        jax.ShapeDtypeStruct((_N, _D), jnp.float32),
    ),
)


def _final_body(h, aggp, dinv, W, b, out):
    agg = jnp.sum(aggp[:, : _N, :], axis=0)
    t = h[...] - _ALPHA * dinv[...] * agg
    out[...] = jnp.dot(t, W[...], preferred_element_type=jnp.float32) + b[...]


_final_call = pl.pallas_call(
    _final_body,
    out_shape=jax.ShapeDtypeStruct((_N, _D), jnp.float32),
)


def kernel(x, edge_index, W1, b1, W2, b2, W3, b3, g1, be1, g2, be2):
    row = edge_index[0]
    col = edge_index[1]
    pad = _EPAD - _E
    rowp = jnp.concatenate(
        [row, jnp.full((pad,), _DUMMY, jnp.int32)]).reshape(_NW, _CPW, _CH)
    colp = jnp.concatenate(
        [col, jnp.zeros((pad,), jnp.int32)]).reshape(_NW, _CPW, _CH)

    degp = _deg_call(rowp)
    dinv, g = _prep_call(degp, x)

    aggp = _agg_call(g, colp, rowp)
    h, g = _dense_call(x, aggp, dinv, W1, b1.reshape(1, _D),
                       g1.reshape(1, _D), be1.reshape(1, _D))
    aggp = _agg_call(g, colp, rowp)
    h, g = _dense_call(h, aggp, dinv, W2, b2.reshape(1, _D),
                       g2.reshape(1, _D), be2.reshape(1, _D))
    aggp = _agg_call(g, colp, rowp)
    return _final_call(h, aggp, dinv, W3, b3.reshape(1, _D))
